# Initial kernel scaffold; baseline (speedup 1.0000x reference)
import jax
import jax.numpy as jnp
from jax.experimental import pallas as pl


def kernel(x, W_enc, b_enc, pre_bias):
    raise NotImplementedError("write your pallas kernel here")


# speedup vs baseline: 1.7631x; 1.7631x over previous
import functools

import jax
import jax.numpy as jnp
from jax import lax
from jax.experimental import pallas as pl
from jax.experimental.pallas import tpu as pltpu
from jax.experimental.pallas import tpu_sc as plsc

_INPUT_DIM = 2048
_LATENT_DIM = 32768
_BATCH = 32
_K = 32
_LBLK = 512
_NBLK = _LATENT_DIM // _LBLK

_NC = 2
_NS = 16
_L = 16

_GROUP = 8
_NCHUNK = _LATENT_DIM // _L
_NGROUP = _NCHUNK // _GROUP

_NEG_INF = float("-inf")


def _enc_body(x_ref, w_ref, b_ref, lat_ref, xn_ref):
    j = pl.program_id(0)

    @pl.when(j == 0)
    def _():
        x = x_ref[...]
        mu = jnp.mean(x, axis=1, keepdims=True)
        xc = x - mu
        var = jnp.sum(xc * xc, axis=1, keepdims=True) / (_INPUT_DIM - 1)
        std = jnp.sqrt(var)
        xn_ref[...] = xc / (std + 1e-5)

    xn = xn_ref[...]
    w = w_ref[...]
    acc = lax.dot_general(
        xn, w, (((1,), (1,)), ((), ())),
        preferred_element_type=jnp.float32,
    )
    lat_ref[...] = acc + b_ref[0]


def _encode(x, w, b2):
    return pl.pallas_call(
        _enc_body,
        grid=(_NBLK,),
        in_specs=[
            pl.BlockSpec((_BATCH, _INPUT_DIM), lambda j: (0, 0)),
            pl.BlockSpec((_LBLK, _INPUT_DIM), lambda j: (j, 0)),
            pl.BlockSpec((1, 1, _LBLK), lambda j: (j, 0, 0)),
        ],
        out_specs=pl.BlockSpec((_BATCH, _LBLK), lambda j: (0, j)),
        out_shape=jax.ShapeDtypeStruct((_BATCH, _LATENT_DIM), jnp.float32),
        scratch_shapes=[pltpu.VMEM((_BATCH, _INPUT_DIM), jnp.float32)],
    )(x, w, b2)


def _merge_topk(kv0, kv1, ki0, ki1, sv, si):
    rsv = lax.rev(sv, (0,))
    rsi = lax.rev(si, (0,))
    ge = kv1 >= rsv
    c1 = jnp.where(ge, kv1, rsv)
    c1i = jnp.where(ge, ki1, rsi)
    ge2 = kv0 >= c1
    hi = jnp.where(ge2, kv0, c1)
    hii = jnp.where(ge2, ki0, c1i)
    lo = jnp.where(ge2, c1, kv0)
    loi = jnp.where(ge2, c1i, ki0)
    kv0, ki0 = plsc.sort_key_val(hi, hii, descending=True)
    kv1, ki1 = plsc.sort_key_val(lo, loi, descending=True)
    return kv0, kv1, ki0, ki1


def _sc_body(lat_hbm, w_hbm, pb_hbm, out_hbm, sparse_hbm,
             lat_v, rows_v, outv, pbv, kv_ref, ki_ref, sem0, sem1):
    wid = lax.axis_index("s") * _NC + lax.axis_index("c")

    pltpu.sync_copy(lat_hbm.at[wid], lat_v)
    pltpu.sync_copy(pb_hbm, pbv)

    lane = lax.iota(jnp.int32, _L)
    zeros16 = jnp.zeros((_L,), jnp.float32)
    neg_inf16 = jnp.full((_L,), _NEG_INF, jnp.float32)

    kv_ref[pl.ds(0, _L)] = neg_inf16
    kv_ref[pl.ds(_L, _L)] = neg_inf16
    ki_ref[pl.ds(0, _L)] = jnp.zeros((_L,), jnp.int32)
    ki_ref[pl.ds(_L, _L)] = jnp.zeros((_L,), jnp.int32)

    def group_body(g, t):
        base = g * (_GROUP * _L)
        vs = []
        for j in range(_GROUP):
            vs.append(lat_v[pl.ds(base + j * _L, _L)])
        acc = vs[0]
        for j in range(1, _GROUP):
            acc = jnp.maximum(acc, vs[j])
        for j in range(_GROUP):
            lat_v[pl.ds(base + j * _L, _L)] = zeros16
        hit = plsc.all_reduce_population_count(acc > t)[0] > 0

        def slow(t):
            for j in range(_GROUP):
                v = vs[j]
                chunk_hit = plsc.all_reduce_population_count(v > t)[0] > 0

                def do_merge(t, v=v, j=j):
                    kv0 = kv_ref[pl.ds(0, _L)]
                    kv1 = kv_ref[pl.ds(_L, _L)]
                    ki0 = ki_ref[pl.ds(0, _L)]
                    ki1 = ki_ref[pl.ds(_L, _L)]
                    vidx = base + j * _L + lane
                    sv, si = plsc.sort_key_val(v, vidx, descending=True)
                    kv0, kv1, ki0, ki1 = _merge_topk(kv0, kv1, ki0, ki1, sv, si)
                    kv_ref[pl.ds(0, _L)] = kv0
                    kv_ref[pl.ds(_L, _L)] = kv1
                    ki_ref[pl.ds(0, _L)] = ki0
                    ki_ref[pl.ds(_L, _L)] = ki1
                    return kv1[_L - 1]

                t = lax.cond(chunk_hit, do_merge, lambda t: t, t)
            return t

        return lax.cond(hit, slow, lambda t: t, t)

    lax.fori_loop(0, _NGROUP, group_body, jnp.float32(_NEG_INF))

    kv0 = kv_ref[pl.ds(0, _L)]
    kv1 = kv_ref[pl.ds(_L, _L)]
    ki0 = ki_ref[pl.ds(0, _L)]
    ki1 = ki_ref[pl.ds(_L, _L)]
    plsc.store_scatter(lat_v, [ki0], kv0)
    plsc.store_scatter(lat_v, [ki1], kv1)

    cp_sparse = pltpu.make_async_copy(lat_v, sparse_hbm.at[wid], sem0)
    cp_sparse.start()

    pltpu.make_async_copy(w_hbm.at[ki_ref], rows_v, sem1).start()
    pltpu.make_async_copy(w_hbm.at[ki_ref], rows_v, sem1).wait()

    vals = [kv0[i] for i in range(_L)] + [kv1[i] for i in range(_L)]

    def col_body(jj, _):
        col = jj * _L
        acc = pbv[pl.ds(col, _L)]
        for k in range(_K):
            acc = acc + vals[k] * rows_v[k, pl.ds(col, _L)]
        outv[pl.ds(col, _L)] = acc
        return 0

    lax.fori_loop(0, _INPUT_DIM // _L, col_body, 0)

    pltpu.sync_copy(outv, out_hbm.at[wid])
    cp_sparse.wait()


def _sc_topk_decode(latents, w, pre_bias):
    mesh = plsc.VectorSubcoreMesh(
        core_axis_name="c", subcore_axis_name="s",
        num_cores=_NC, num_subcores=_NS,
    )
    f = pl.kernel(
        _sc_body,
        out_type=[
            jax.ShapeDtypeStruct((_BATCH, _INPUT_DIM), jnp.float32),
            jax.ShapeDtypeStruct((_BATCH, _LATENT_DIM), jnp.float32),
        ],
        mesh=mesh,
        compiler_params=pltpu.CompilerParams(needs_layout_passes=False),
        scratch_types=[
            pltpu.VMEM((_LATENT_DIM,), jnp.float32),
            pltpu.VMEM((_K, _INPUT_DIM), jnp.float32),
            pltpu.VMEM((_INPUT_DIM,), jnp.float32),
            pltpu.VMEM((_INPUT_DIM,), jnp.float32),
            pltpu.VMEM((2 * _L,), jnp.float32),
            pltpu.VMEM((2 * _L,), jnp.int32),
            pltpu.SemaphoreType.DMA,
            pltpu.SemaphoreType.DMA,
        ],
    )
    return f(latents, w, pre_bias)


@jax.jit
def kernel(x, W_enc, b_enc, pre_bias):
    b2 = b_enc.reshape(_NBLK, 1, _LBLK)
    latents = _encode(x, W_enc, b2)
    output, sparse_latents = _sc_topk_decode(latents, W_enc, pre_bias)
    return (output, sparse_latents)


# speedup vs baseline: 1.9644x; 1.1142x over previous
import functools

import jax
import jax.numpy as jnp
from jax import lax
from jax.experimental import pallas as pl
from jax.experimental.pallas import tpu as pltpu
from jax.experimental.pallas import tpu_sc as plsc

_INPUT_DIM = 2048
_LATENT_DIM = 32768
_BATCH = 32
_K = 32
_LBLK = 1024
_NBLK = _LATENT_DIM // _LBLK

_NC = 2
_NS = 16
_L = 16

_GROUP = 8
_NCHUNK = _LATENT_DIM // _L
_NGROUP = _NCHUNK // _GROUP

_NEG_INF = float("-inf")


def _enc_body(x_ref, w_ref, b_ref, lat_ref, xn_ref):
    j = pl.program_id(0)

    @pl.when(j == 0)
    def _():
        x = x_ref[...]
        mu = jnp.mean(x, axis=1, keepdims=True)
        xc = x - mu
        var = jnp.sum(xc * xc, axis=1, keepdims=True) / (_INPUT_DIM - 1)
        std = jnp.sqrt(var)
        xn_ref[...] = xc / (std + 1e-5)

    xn = xn_ref[...]
    w = w_ref[...]
    acc = lax.dot_general(
        xn, w, (((1,), (1,)), ((), ())),
        preferred_element_type=jnp.float32,
    )
    lat_ref[...] = acc + b_ref[0]


def _encode(x, w, b2):
    return pl.pallas_call(
        _enc_body,
        grid=(_NBLK,),
        in_specs=[
            pl.BlockSpec((_BATCH, _INPUT_DIM), lambda j: (0, 0)),
            pl.BlockSpec((_LBLK, _INPUT_DIM), lambda j: (j, 0)),
            pl.BlockSpec((1, 1, _LBLK), lambda j: (j, 0, 0)),
        ],
        out_specs=pl.BlockSpec((_BATCH, _LBLK), lambda j: (0, j)),
        out_shape=jax.ShapeDtypeStruct((_BATCH, _LATENT_DIM), jnp.float32),
        scratch_shapes=[pltpu.VMEM((_BATCH, _INPUT_DIM), jnp.float32)],
    )(x, w, b2)


def _merge_topk(kv0, kv1, ki0, ki1, sv, si):
    rsv = lax.rev(sv, (0,))
    rsi = lax.rev(si, (0,))
    ge = kv1 >= rsv
    c1 = jnp.where(ge, kv1, rsv)
    c1i = jnp.where(ge, ki1, rsi)
    ge2 = kv0 >= c1
    hi = jnp.where(ge2, kv0, c1)
    hii = jnp.where(ge2, ki0, c1i)
    lo = jnp.where(ge2, c1, kv0)
    loi = jnp.where(ge2, c1i, ki0)
    kv0, ki0 = plsc.sort_key_val(hi, hii, descending=True)
    kv1, ki1 = plsc.sort_key_val(lo, loi, descending=True)
    return kv0, kv1, ki0, ki1


def _sc_body(lat_hbm, w_hbm, pb_hbm, out_hbm, sparse_hbm,
             lat_v, rows_v, outv, pbv, kv_ref, ki_ref, sem0, sem1):
    wid = lax.axis_index("s") * _NC + lax.axis_index("c")

    with jax.named_scope("sc_dma_in"):
        pltpu.sync_copy(lat_hbm.at[wid], lat_v)
        pltpu.sync_copy(pb_hbm, pbv)

    lane = lax.iota(jnp.int32, _L)
    zeros16 = jnp.zeros((_L,), jnp.float32)
    neg_inf16 = jnp.full((_L,), _NEG_INF, jnp.float32)

    kv_ref[pl.ds(0, _L)] = neg_inf16
    kv_ref[pl.ds(_L, _L)] = neg_inf16
    ki_ref[pl.ds(0, _L)] = jnp.zeros((_L,), jnp.int32)
    ki_ref[pl.ds(_L, _L)] = jnp.zeros((_L,), jnp.int32)

    def group_body(g, t):
        base = g * (_GROUP * _L)
        vs = []
        for j in range(_GROUP):
            vs.append(lat_v[pl.ds(base + j * _L, _L)])
        acc = vs[0]
        for j in range(1, _GROUP):
            acc = jnp.maximum(acc, vs[j])
        for j in range(_GROUP):
            lat_v[pl.ds(base + j * _L, _L)] = zeros16
        hit = plsc.all_reduce_population_count(acc > t)[0] > 0

        def slow(t):
            for j in range(_GROUP):
                v = vs[j]
                chunk_hit = plsc.all_reduce_population_count(v > t)[0] > 0

                def do_merge(t, v=v, j=j):
                    kv0 = kv_ref[pl.ds(0, _L)]
                    kv1 = kv_ref[pl.ds(_L, _L)]
                    ki0 = ki_ref[pl.ds(0, _L)]
                    ki1 = ki_ref[pl.ds(_L, _L)]
                    vidx = base + j * _L + lane
                    sv, si = plsc.sort_key_val(v, vidx, descending=True)
                    kv0, kv1, ki0, ki1 = _merge_topk(kv0, kv1, ki0, ki1, sv, si)
                    kv_ref[pl.ds(0, _L)] = kv0
                    kv_ref[pl.ds(_L, _L)] = kv1
                    ki_ref[pl.ds(0, _L)] = ki0
                    ki_ref[pl.ds(_L, _L)] = ki1
                    return kv1[_L - 1]

                t = lax.cond(chunk_hit, do_merge, lambda t: t, t)
            return t

        return lax.cond(hit, slow, lambda t: t, t)

    with jax.named_scope("sc_topk_scan"):
        lax.fori_loop(0, _NGROUP, group_body, jnp.float32(_NEG_INF))

    kv0 = kv_ref[pl.ds(0, _L)]
    kv1 = kv_ref[pl.ds(_L, _L)]
    ki0 = ki_ref[pl.ds(0, _L)]
    ki1 = ki_ref[pl.ds(_L, _L)]
    with jax.named_scope("sc_scatter_gather"):
        plsc.store_scatter(lat_v, [ki0], kv0)
        plsc.store_scatter(lat_v, [ki1], kv1)

        cp_sparse = pltpu.make_async_copy(lat_v, sparse_hbm.at[wid], sem0)
        cp_sparse.start()

        gather = pltpu.make_async_copy(w_hbm.at[ki_ref], rows_v, sem1)
        gather.start()
        gather.wait()

    vals = [kv0[i] for i in range(_L)] + [kv1[i] for i in range(_L)]

    def col_body(jj, _):
        col = jj * _L
        acc = pbv[pl.ds(col, _L)]
        for k in range(_K):
            acc = acc + vals[k] * rows_v[k, pl.ds(col, _L)]
        outv[pl.ds(col, _L)] = acc
        return 0

    with jax.named_scope("sc_decode"):
        lax.fori_loop(0, _INPUT_DIM // _L, col_body, 0)

    with jax.named_scope("sc_dma_out"):
        pltpu.sync_copy(outv, out_hbm.at[wid])
        cp_sparse.wait()


def _sc_topk_decode(latents, w, pre_bias):
    mesh = plsc.VectorSubcoreMesh(
        core_axis_name="c", subcore_axis_name="s",
        num_cores=_NC, num_subcores=_NS,
    )
    f = pl.kernel(
        _sc_body,
        out_type=[
            jax.ShapeDtypeStruct((_BATCH, _INPUT_DIM), jnp.float32),
            jax.ShapeDtypeStruct((_BATCH, _LATENT_DIM), jnp.float32),
        ],
        mesh=mesh,
        compiler_params=pltpu.CompilerParams(needs_layout_passes=False),
        scratch_types=[
            pltpu.VMEM((_LATENT_DIM,), jnp.float32),
            pltpu.VMEM((_K, _INPUT_DIM), jnp.float32),
            pltpu.VMEM((_INPUT_DIM,), jnp.float32),
            pltpu.VMEM((_INPUT_DIM,), jnp.float32),
            pltpu.VMEM((2 * _L,), jnp.float32),
            pltpu.VMEM((2 * _L,), jnp.int32),
            pltpu.SemaphoreType.DMA,
            pltpu.SemaphoreType.DMA,
        ],
    )
    return f(latents, w, pre_bias)


@jax.jit
def kernel(x, W_enc, b_enc, pre_bias):
    b2 = b_enc.reshape(_NBLK, 1, _LBLK)
    latents = _encode(x, W_enc, b2)
    output, sparse_latents = _sc_topk_decode(latents, W_enc, pre_bias)
    return (output, sparse_latents)


# speedup vs baseline: 2.0433x; 1.0402x over previous
import jax
import jax.numpy as jnp
from jax import lax
from jax.experimental import pallas as pl
from jax.experimental.pallas import tpu as pltpu
from jax.experimental.pallas import tpu_sc as plsc

_INPUT_DIM = 2048
_LATENT_DIM = 32768
_BATCH = 32
_K = 32
_LBLK = 2048
_NBLK = _LATENT_DIM // _LBLK

_NC = 2
_NS = 16
_L = 16

_GSZ = 128
_NGRP = _LATENT_DIM // _GSZ
_GPB = _LBLK // _GSZ

_NEG_INF = float("-inf")


def _enc_body(x_ref, w_ref, b_ref, lat_ref, gm_ref, xn_ref):
    j = pl.program_id(0)

    @pl.when(j == 0)
    def _():
        x = x_ref[...]
        mu = jnp.mean(x, axis=1, keepdims=True)
        xc = x - mu
        var = jnp.sum(xc * xc, axis=1, keepdims=True) / (_INPUT_DIM - 1)
        std = jnp.sqrt(var)
        xn_ref[...] = xc / (std + 1e-5)

    xn = xn_ref[...]
    w = w_ref[...]
    acc = lax.dot_general(
        xn, w, (((1,), (1,)), ((), ())),
        preferred_element_type=jnp.float32,
    )
    lat = acc + b_ref[0]
    lat_ref[...] = lat
    gm = jnp.max(lat.reshape(_BATCH, _GPB, _GSZ), axis=2)
    gm_ref[...] = gm.reshape(_BATCH, 1, 1, _GPB)


def _encode(x, w, b2):
    return pl.pallas_call(
        _enc_body,
        grid=(_NBLK,),
        in_specs=[
            pl.BlockSpec((_BATCH, _INPUT_DIM), lambda j: (0, 0)),
            pl.BlockSpec((_LBLK, _INPUT_DIM), lambda j: (j, 0)),
            pl.BlockSpec((1, 1, _LBLK), lambda j: (j, 0, 0)),
        ],
        out_specs=[
            pl.BlockSpec((_BATCH, _LBLK), lambda j: (0, j)),
            pl.BlockSpec((_BATCH, 1, 1, _GPB), lambda j: (0, j, 0, 0)),
        ],
        out_shape=[
            jax.ShapeDtypeStruct((_BATCH, _LATENT_DIM), jnp.float32),
            jax.ShapeDtypeStruct((_BATCH, _NBLK, 1, _GPB), jnp.float32),
        ],
        scratch_shapes=[pltpu.VMEM((_BATCH, _INPUT_DIM), jnp.float32)],
    )(x, w, b2)


def _merge_topk(kv0, kv1, ki0, ki1, sv, si):
    rsv = lax.rev(sv, (0,))
    rsi = lax.rev(si, (0,))
    ge = kv1 >= rsv
    c1 = jnp.where(ge, kv1, rsv)
    c1i = jnp.where(ge, ki1, rsi)
    ge2 = kv0 >= c1
    hi = jnp.where(ge2, kv0, c1)
    hii = jnp.where(ge2, ki0, c1i)
    lo = jnp.where(ge2, c1, kv0)
    loi = jnp.where(ge2, c1i, ki0)
    kv0, ki0 = plsc.sort_key_val(hi, hii, descending=True)
    kv1, ki1 = plsc.sort_key_val(lo, loi, descending=True)
    return kv0, kv1, ki0, ki1


def _sc_body(latf_hbm, gmax_hbm, w_hbm, pb_hbm, out_hbm, sparse_hbm,
             zbuf, grp_rows, rows_v, outv, pbv, gmaxv,
             kv_ref, ki_ref, gi_ref, sem0, sem1, sem2, sem3):
    wid = lax.axis_index("s") * _NC + lax.axis_index("c")

    lane = lax.iota(jnp.int32, _L)
    zeros16 = jnp.zeros((_L,), jnp.float32)
    neg_inf16 = jnp.full((_L,), _NEG_INF, jnp.float32)
    izeros16 = jnp.zeros((_L,), jnp.int32)

    with jax.named_scope("sc_dma_in"):
        cp_gm = pltpu.make_async_copy(gmax_hbm.at[wid], gmaxv, sem0)
        cp_gm.start()
        cp_pb = pltpu.make_async_copy(pb_hbm, pbv, sem1)
        cp_pb.start()

    with jax.named_scope("sc_zero"):
        def zero_body(i, c):
            base = i * (4 * _L)
            for u in range(4):
                zbuf[pl.ds(base + u * _L, _L)] = zeros16
            return c

        lax.fori_loop(0, _LATENT_DIM // (4 * _L), zero_body, 0)

    with jax.named_scope("sc_phase_a"):
        cp_gm.wait()
        gv0, gv1 = neg_inf16, neg_inf16
        gi0, gi1 = izeros16, izeros16
        for c in range(_NGRP // _L):
            v = gmaxv[pl.ds(c * _L, _L)]
            sv, si = plsc.sort_key_val(v, c * _L + lane, descending=True)
            gv0, gv1, gi0, gi1 = _merge_topk(gv0, gv1, gi0, gi1, sv, si)
        gi_ref[pl.ds(0, _L)] = gi0 + wid * _NGRP
        gi_ref[pl.ds(_L, _L)] = gi1 + wid * _NGRP
        g_scal = [gi0[i] for i in range(_L)] + [gi1[i] for i in range(_L)]

    with jax.named_scope("sc_group_gather"):
        cp_grp = pltpu.make_async_copy(latf_hbm.at[gi_ref], grp_rows, sem2)
        cp_grp.start()
        cp_grp.wait()

    with jax.named_scope("sc_phase_b"):
        kv_ref[pl.ds(0, _L)] = neg_inf16
        kv_ref[pl.ds(_L, _L)] = neg_inf16
        ki_ref[pl.ds(0, _L)] = izeros16
        ki_ref[pl.ds(_L, _L)] = izeros16

        def scan_group(k, g):
            def chunk_body(c, t):
                v = grp_rows[k, pl.ds(c * _L, _L)]
                hit = plsc.all_reduce_population_count(v > t)[0] > 0

                def do_merge(t):
                    kv0 = kv_ref[pl.ds(0, _L)]
                    kv1 = kv_ref[pl.ds(_L, _L)]
                    ki0 = ki_ref[pl.ds(0, _L)]
                    ki1 = ki_ref[pl.ds(_L, _L)]
                    vidx = g * _GSZ + c * _L + lane
                    sv, si = plsc.sort_key_val(v, vidx, descending=True)
                    kv0, kv1, ki0, ki1 = _merge_topk(kv0, kv1, ki0, ki1, sv, si)
                    kv_ref[pl.ds(0, _L)] = kv0
                    kv_ref[pl.ds(_L, _L)] = kv1
                    ki_ref[pl.ds(0, _L)] = ki0
                    ki_ref[pl.ds(_L, _L)] = ki1
                    return kv1[_L - 1]

                return lax.cond(hit, do_merge, lambda t: t, t)

            return lambda t: lax.fori_loop(0, _GSZ // _L, chunk_body, t)

        t = jnp.float32(_NEG_INF)
        for k in range(_K):
            t = scan_group(k, g_scal[k])(t)

    with jax.named_scope("sc_scatter"):
        kv0 = kv_ref[pl.ds(0, _L)]
        kv1 = kv_ref[pl.ds(_L, _L)]
        ki0 = ki_ref[pl.ds(0, _L)]
        ki1 = ki_ref[pl.ds(_L, _L)]
        plsc.store_scatter(zbuf, [ki0], kv0)
        plsc.store_scatter(zbuf, [ki1], kv1)

        gA = pltpu.make_async_copy(
            w_hbm.at[ki_ref.at[pl.ds(0, _L)]], rows_v.at[pl.ds(0, _L)], sem1)
        gB = pltpu.make_async_copy(
            w_hbm.at[ki_ref.at[pl.ds(_L, _L)]], rows_v.at[pl.ds(_L, _L)], sem3)
        gA.start()
        gB.start()

        cp_sparse = pltpu.make_async_copy(zbuf, sparse_hbm.at[wid], sem0)
        cp_sparse.start()

    with jax.named_scope("sc_decode"):
        cp_pb.wait()
        vals = [kv0[i] for i in range(_L)] + [kv1[i] for i in range(_L)]

        def make_col_body(k0, k1):
            def col_body(jj, c):
                col = jj * (2 * _L)
                for u in range(2):
                    cu = col + u * _L
                    acc = outv[pl.ds(cu, _L)] if k0 else pbv[pl.ds(cu, _L)]
                    for k in range(k0, k1):
                        acc = acc + vals[k] * rows_v[k, pl.ds(cu, _L)]
                    outv[pl.ds(cu, _L)] = acc
                return c
            return col_body

        gA.wait()
        lax.fori_loop(0, _INPUT_DIM // (2 * _L), make_col_body(0, _L), 0)
        gB.wait()
        lax.fori_loop(0, _INPUT_DIM // (2 * _L), make_col_body(_L, _K), 0)

    with jax.named_scope("sc_dma_out"):
        pltpu.sync_copy(outv, out_hbm.at[wid])
        cp_sparse.wait()


def _sc_topk_decode(latents, gmax, w, pre_bias):
    latf = latents.reshape(_BATCH * _NGRP, _GSZ)
    mesh = plsc.VectorSubcoreMesh(
        core_axis_name="c", subcore_axis_name="s",
        num_cores=_NC, num_subcores=_NS,
    )
    f = pl.kernel(
        _sc_body,
        out_type=[
            jax.ShapeDtypeStruct((_BATCH, _INPUT_DIM), jnp.float32),
            jax.ShapeDtypeStruct((_BATCH, _LATENT_DIM), jnp.float32),
        ],
        mesh=mesh,
        compiler_params=pltpu.CompilerParams(needs_layout_passes=False),
        scratch_types=[
            pltpu.VMEM((_LATENT_DIM,), jnp.float32),
            pltpu.VMEM((_K, _GSZ), jnp.float32),
            pltpu.VMEM((_K, _INPUT_DIM), jnp.float32),
            pltpu.VMEM((_INPUT_DIM,), jnp.float32),
            pltpu.VMEM((_INPUT_DIM,), jnp.float32),
            pltpu.VMEM((_NGRP,), jnp.float32),
            pltpu.VMEM((2 * _L,), jnp.float32),
            pltpu.VMEM((2 * _L,), jnp.int32),
            pltpu.VMEM((2 * _L,), jnp.int32),
            pltpu.SemaphoreType.DMA,
            pltpu.SemaphoreType.DMA,
            pltpu.SemaphoreType.DMA,
            pltpu.SemaphoreType.DMA,
        ],
    )
    return f(latf, gmax, w, pre_bias)


@jax.jit
def kernel(x, W_enc, b_enc, pre_bias):
    b2 = b_enc.reshape(_NBLK, 1, _LBLK)
    latents, gmax4 = _encode(x, W_enc, b2)
    gmax = gmax4.reshape(_BATCH, _NGRP)
    output, sparse_latents = _sc_topk_decode(latents, gmax, W_enc, pre_bias)
    return (output, sparse_latents)


# speedup vs baseline: 2.2248x; 1.0888x over previous
import jax
import jax.numpy as jnp
from jax import lax
from jax.experimental import pallas as pl
from jax.experimental.pallas import tpu as pltpu
from jax.experimental.pallas import tpu_sc as plsc

_INPUT_DIM = 2048
_LATENT_DIM = 32768
_BATCH = 32
_K = 32
_LBLK = 2048
_NBLK = _LATENT_DIM // _LBLK

_NC = 2
_NS = 16
_L = 16

_GSZ = 128
_NGRP = _LATENT_DIM // _GSZ
_GPB = _LBLK // _GSZ

_NEG_INF = float("-inf")


def _enc_body(x_ref, w_ref, b_ref, lat_ref, gm_ref, xn_ref):
    j = pl.program_id(0)

    @pl.when(j == 0)
    def _():
        x = x_ref[...]
        mu = jnp.mean(x, axis=1, keepdims=True)
        xc = x - mu
        var = jnp.sum(xc * xc, axis=1, keepdims=True) / (_INPUT_DIM - 1)
        std = jnp.sqrt(var)
        xn_ref[...] = xc / (std + 1e-5)

    xn = xn_ref[...]
    w = w_ref[...]
    acc = lax.dot_general(
        xn, w, (((1,), (1,)), ((), ())),
        preferred_element_type=jnp.float32,
    )
    lat = acc + b_ref[0]
    lat_ref[...] = lat
    gm = jnp.max(lat.reshape(_BATCH, _GPB, _GSZ), axis=2)
    gm_ref[...] = gm.reshape(_BATCH, 1, 1, _GPB)


def _encode(x, w, b2):
    return pl.pallas_call(
        _enc_body,
        grid=(_NBLK,),
        in_specs=[
            pl.BlockSpec((_BATCH, _INPUT_DIM), lambda j: (0, 0)),
            pl.BlockSpec((_LBLK, _INPUT_DIM), lambda j: (j, 0)),
            pl.BlockSpec((1, 1, _LBLK), lambda j: (j, 0, 0)),
        ],
        out_specs=[
            pl.BlockSpec((_BATCH, _LBLK), lambda j: (0, j)),
            pl.BlockSpec((_BATCH, 1, 1, _GPB), lambda j: (0, j, 0, 0)),
        ],
        out_shape=[
            jax.ShapeDtypeStruct((_BATCH, _LATENT_DIM), jnp.float32),
            jax.ShapeDtypeStruct((_BATCH, _NBLK, 1, _GPB), jnp.float32),
        ],
        scratch_shapes=[pltpu.VMEM((_BATCH, _INPUT_DIM), jnp.float32)],
    )(x, w, b2)


def _merge_topk(kv0, kv1, ki0, ki1, sv, si):
    rsv = lax.rev(sv, (0,))
    rsi = lax.rev(si, (0,))
    ge = kv1 >= rsv
    c1 = jnp.where(ge, kv1, rsv)
    c1i = jnp.where(ge, ki1, rsi)
    ge2 = kv0 >= c1
    hi = jnp.where(ge2, kv0, c1)
    hii = jnp.where(ge2, ki0, c1i)
    lo = jnp.where(ge2, c1, kv0)
    loi = jnp.where(ge2, c1i, ki0)
    kv0, ki0 = plsc.sort_key_val(hi, hii, descending=True)
    kv1, ki1 = plsc.sort_key_val(lo, loi, descending=True)
    return kv0, kv1, ki0, ki1


def _sc_body(latf_hbm, gmax_hbm, w_hbm, pb_hbm, out_hbm, sparse_hbm,
             zbuf, grp_rows, rows_v, outv, pbv, gmaxv,
             cand_v, cand_i, ki_ref, gi_ref, sem0, sem1, sem2, sem3):
    wid = lax.axis_index("s") * _NC + lax.axis_index("c")

    lane = lax.iota(jnp.int32, _L)
    zeros16 = jnp.zeros((_L,), jnp.float32)
    neg_inf16 = jnp.full((_L,), _NEG_INF, jnp.float32)
    izeros16 = jnp.zeros((_L,), jnp.int32)

    with jax.named_scope("sc_dma_in"):
        cp_gm = pltpu.make_async_copy(gmax_hbm.at[wid], gmaxv, sem0)
        cp_gm.start()
        cp_pb = pltpu.make_async_copy(pb_hbm, pbv, sem1)
        cp_pb.start()

    with jax.named_scope("sc_phase_a"):
        cp_gm.wait()
        gv0, gv1 = neg_inf16, neg_inf16
        gi0, gi1 = izeros16, izeros16
        for c in range(_NGRP // _L):
            v = gmaxv[pl.ds(c * _L, _L)]
            sv, si = plsc.sort_key_val(v, c * _L + lane, descending=True)
            gv0, gv1, gi0, gi1 = _merge_topk(gv0, gv1, gi0, gi1, sv, si)
        gi_ref[pl.ds(0, _L)] = gi0 + wid * _NGRP
        gi_ref[pl.ds(_L, _L)] = gi1 + wid * _NGRP
        g_scal = [gi0[i] for i in range(_L)] + [gi1[i] for i in range(_L)]
        t0g = gv1[_L - 1]

    with jax.named_scope("sc_group_gather"):
        cp_grp = pltpu.make_async_copy(latf_hbm.at[gi_ref], grp_rows, sem2)
        cp_grp.start()
        cp_grp.wait()

    with jax.named_scope("sc_phase_b"):
        cnt = jnp.int32(0)
        for k in range(_K):
            g = g_scal[k]

            def chunk_body(c, cnt, k=k, g=g):
                v = grp_rows[k, pl.ds(c * _L, _L)]
                for u in range(_GSZ // _L):
                    zbuf[pl.ds(k * 1024 + c * _GSZ + u * _L, _L)] = zeros16
                m = v >= t0g
                iv = g * _GSZ + c * _L + lane
                plsc.store_compressed(cand_v.at[pl.ds(cnt, _L)], v, mask=m)
                plsc.store_compressed(cand_i.at[pl.ds(cnt, _L)], iv, mask=m)
                return cnt + plsc.all_reduce_population_count(m)[0]

            cnt = lax.fori_loop(0, _GSZ // _L, chunk_body, cnt)

        def p2_body(i, carry):
            kv0, kv1, ki0, ki1 = carry
            v = cand_v[pl.ds(i * _L, _L)]
            iv = cand_i[pl.ds(i * _L, _L)]
            valid = (i * _L + lane) < cnt
            v = jnp.where(valid, v, neg_inf16)
            sv, si = plsc.sort_key_val(v, iv, descending=True)
            return _merge_topk(kv0, kv1, ki0, ki1, sv, si)

        nv = (cnt + _L - 1) // _L
        kv0, kv1, ki0, ki1 = lax.fori_loop(
            0, nv, p2_body, (neg_inf16, neg_inf16, izeros16, izeros16))

    with jax.named_scope("sc_scatter"):
        ki_ref[pl.ds(0, _L)] = ki0
        ki_ref[pl.ds(_L, _L)] = ki1
        plsc.store_scatter(zbuf, [ki0], kv0)
        plsc.store_scatter(zbuf, [ki1], kv1)

        cp_w = pltpu.make_async_copy(w_hbm.at[ki_ref], rows_v, sem3)
        cp_w.start()

        cp_sparse = pltpu.make_async_copy(zbuf, sparse_hbm.at[wid], sem0)
        cp_sparse.start()

    with jax.named_scope("sc_decode"):
        cp_pb.wait()
        cp_w.wait()
        vals = [kv0[i] for i in range(_L)] + [kv1[i] for i in range(_L)]

        def col_body(jj, c):
            col = jj * (2 * _L)
            for u in range(2):
                cu = col + u * _L
                acc = pbv[pl.ds(cu, _L)]
                for k in range(_K):
                    acc = acc + vals[k] * rows_v[k, pl.ds(cu, _L)]
                outv[pl.ds(cu, _L)] = acc
            return c

        lax.fori_loop(0, _INPUT_DIM // (2 * _L), col_body, 0)

    with jax.named_scope("sc_dma_out"):
        pltpu.sync_copy(outv, out_hbm.at[wid])
        cp_sparse.wait()


def _sc_topk_decode(latents, gmax, w, pre_bias):
    latf = latents.reshape(_BATCH * _NGRP, _GSZ)
    mesh = plsc.VectorSubcoreMesh(
        core_axis_name="c", subcore_axis_name="s",
        num_cores=_NC, num_subcores=_NS,
    )
    f = pl.kernel(
        _sc_body,
        out_type=[
            jax.ShapeDtypeStruct((_BATCH, _INPUT_DIM), jnp.float32),
            jax.ShapeDtypeStruct((_BATCH, _LATENT_DIM), jnp.float32),
        ],
        mesh=mesh,
        compiler_params=pltpu.CompilerParams(needs_layout_passes=False),
        scratch_types=[
            pltpu.VMEM((_LATENT_DIM,), jnp.float32),
            pltpu.VMEM((_K, _GSZ), jnp.float32),
            pltpu.VMEM((_K, _INPUT_DIM), jnp.float32),
            pltpu.VMEM((_INPUT_DIM,), jnp.float32),
            pltpu.VMEM((_INPUT_DIM,), jnp.float32),
            pltpu.VMEM((_NGRP,), jnp.float32),
            pltpu.VMEM((_K * _GSZ + _L,), jnp.float32),
            pltpu.VMEM((_K * _GSZ + _L,), jnp.int32),
            pltpu.VMEM((2 * _L,), jnp.int32),
            pltpu.VMEM((2 * _L,), jnp.int32),
            pltpu.SemaphoreType.DMA,
            pltpu.SemaphoreType.DMA,
            pltpu.SemaphoreType.DMA,
            pltpu.SemaphoreType.DMA,
        ],
    )
    return f(latf, gmax, w, pre_bias)


@jax.jit
def kernel(x, W_enc, b_enc, pre_bias):
    b2 = b_enc.reshape(_NBLK, 1, _LBLK)
    latents, gmax4 = _encode(x, W_enc, b2)
    gmax = gmax4.reshape(_BATCH, _NGRP)
    output, sparse_latents = _sc_topk_decode(latents, gmax, W_enc, pre_bias)
    return (output, sparse_latents)


# speedup vs baseline: 2.2680x; 1.0194x over previous
import jax
import jax.numpy as jnp
from jax import lax
from jax.experimental import pallas as pl
from jax.experimental.pallas import tpu as pltpu
from jax.experimental.pallas import tpu_sc as plsc

_INPUT_DIM = 2048
_LATENT_DIM = 32768
_BATCH = 32
_K = 32
_LBLK = 2048
_NBLK = _LATENT_DIM // _LBLK

_NC = 2
_NS = 16
_L = 16

_GSZ = 128
_NGRP = _LATENT_DIM // _GSZ
_GPB = _LBLK // _GSZ

_NEG_INF = float("-inf")


def _enc_body(x_ref, w_ref, b_ref, lat_ref, gm_ref, xn_ref):
    j = pl.program_id(0)

    @pl.when(j == 0)
    def _():
        x = x_ref[...]
        mu = jnp.mean(x, axis=1, keepdims=True)
        xc = x - mu
        var = jnp.sum(xc * xc, axis=1, keepdims=True) / (_INPUT_DIM - 1)
        std = jnp.sqrt(var)
        xn_ref[...] = xc / (std + 1e-5)

    xn = xn_ref[...]
    w = w_ref[...]
    acc = lax.dot_general(
        xn, w, (((1,), (1,)), ((), ())),
        preferred_element_type=jnp.float32,
    )
    lat = acc + b_ref[0]
    lat_ref[...] = lat
    gm = jnp.max(lat.reshape(_BATCH, _GPB, _GSZ), axis=2)
    gm_ref[...] = gm.reshape(_BATCH, 1, 1, _GPB)


def _encode(x, w, b2):
    return pl.pallas_call(
        _enc_body,
        grid=(_NBLK,),
        in_specs=[
            pl.BlockSpec((_BATCH, _INPUT_DIM), lambda j: (0, 0)),
            pl.BlockSpec((_LBLK, _INPUT_DIM), lambda j: (j, 0)),
            pl.BlockSpec((1, 1, _LBLK), lambda j: (j, 0, 0)),
        ],
        out_specs=[
            pl.BlockSpec((_BATCH, _LBLK), lambda j: (0, j)),
            pl.BlockSpec((_BATCH, 1, 1, _GPB), lambda j: (0, j, 0, 0)),
        ],
        out_shape=[
            jax.ShapeDtypeStruct((_BATCH, _LATENT_DIM), jnp.float32),
            jax.ShapeDtypeStruct((_BATCH, _NBLK, 1, _GPB), jnp.float32),
        ],
        scratch_shapes=[pltpu.VMEM((_BATCH, _INPUT_DIM), jnp.float32)],
    )(x, w, b2)


def _merge_topk(kv0, kv1, ki0, ki1, sv, si):
    rsv = lax.rev(sv, (0,))
    rsi = lax.rev(si, (0,))
    ge = kv1 >= rsv
    c1 = jnp.where(ge, kv1, rsv)
    c1i = jnp.where(ge, ki1, rsi)
    ge2 = kv0 >= c1
    hi = jnp.where(ge2, kv0, c1)
    hii = jnp.where(ge2, ki0, c1i)
    lo = jnp.where(ge2, c1, kv0)
    loi = jnp.where(ge2, c1i, ki0)
    kv0, ki0 = plsc.sort_key_val(hi, hii, descending=True)
    kv1, ki1 = plsc.sort_key_val(lo, loi, descending=True)
    return kv0, kv1, ki0, ki1


def _sc_body(latf_hbm, gmax_hbm, w_hbm, pb_hbm, out_hbm, sparse_hbm,
             zbuf, grp_rows, rows_v, outv, pbv, gmaxv,
             cand_v, cand_i, ki_ref, gi_ref, sem0, sem1, sem2, sem3):
    wid = lax.axis_index("s") * _NC + lax.axis_index("c")

    lane = lax.iota(jnp.int32, _L)
    zeros16 = jnp.zeros((_L,), jnp.float32)
    neg_inf16 = jnp.full((_L,), _NEG_INF, jnp.float32)
    izeros16 = jnp.zeros((_L,), jnp.int32)

    with jax.named_scope("sc_dma_in"):
        cp_gm = pltpu.make_async_copy(gmax_hbm.at[wid], gmaxv, sem0)
        cp_gm.start()
        cp_pb = pltpu.make_async_copy(pb_hbm, pbv, sem1)
        cp_pb.start()

    with jax.named_scope("sc_phase_a"):
        cp_gm.wait()
        gv0, gv1 = neg_inf16, neg_inf16
        gi0, gi1 = izeros16, izeros16
        for c in range(_NGRP // _L):
            v = gmaxv[pl.ds(c * _L, _L)]
            sv, si = plsc.sort_key_val(v, c * _L + lane, descending=True)
            gv0, gv1, gi0, gi1 = _merge_topk(gv0, gv1, gi0, gi1, sv, si)
        gi_ref[pl.ds(0, _L)] = gi0 + wid * _NGRP
        gi_ref[pl.ds(_L, _L)] = gi1 + wid * _NGRP
        g_scal = [gi0[i] for i in range(_L)] + [gi1[i] for i in range(_L)]
        t0g = gv1[_L - 1]

    with jax.named_scope("sc_group_gather"):
        cp_grp = pltpu.make_async_copy(latf_hbm.at[gi_ref], grp_rows, sem2)
        cp_grp.start()
        cp_grp.wait()

    with jax.named_scope("sc_phase_b"):
        cnt = jnp.int32(0)
        for k in range(_K):
            g = g_scal[k]

            def chunk_body(c, cnt, k=k, g=g):
                v = grp_rows[k, pl.ds(c * _L, _L)]
                for u in range(_GSZ // _L):
                    zbuf[pl.ds(k * 1024 + c * _GSZ + u * _L, _L)] = zeros16
                m = v >= t0g
                iv = g * _GSZ + c * _L + lane
                plsc.store_compressed(cand_v.at[pl.ds(cnt, _L)], v, mask=m)
                plsc.store_compressed(cand_i.at[pl.ds(cnt, _L)], iv, mask=m)
                return cnt + plsc.all_reduce_population_count(m)[0]

            cnt = lax.fori_loop(0, _GSZ // _L, chunk_body, cnt)

        def p2_body(i, carry):
            kv0, kv1, ki0, ki1 = carry
            v = cand_v[pl.ds(i * _L, _L)]
            iv = cand_i[pl.ds(i * _L, _L)]
            valid = (i * _L + lane) < cnt
            v = jnp.where(valid, v, neg_inf16)
            sv, si = plsc.sort_key_val(v, iv, descending=True)
            return _merge_topk(kv0, kv1, ki0, ki1, sv, si)

        nv = (cnt + _L - 1) // _L
        kv0, kv1, ki0, ki1 = lax.fori_loop(
            0, nv, p2_body, (neg_inf16, neg_inf16, izeros16, izeros16))

    with jax.named_scope("sc_scatter"):
        ki_ref[pl.ds(0, _L)] = ki0
        ki_ref[pl.ds(_L, _L)] = ki1
        plsc.store_scatter(zbuf, [ki0], kv0)
        plsc.store_scatter(zbuf, [ki1], kv1)

        cp_w = pltpu.make_async_copy(w_hbm.at[ki_ref], rows_v, sem3)
        cp_w.start()

        cp_sparse = pltpu.make_async_copy(zbuf, sparse_hbm.at[wid], sem0)
        cp_sparse.start()

    with jax.named_scope("sc_gather_wait"):
        cp_pb.wait()
        cp_w.wait()

    with jax.named_scope("sc_decode"):
        vals = [kv0[i] for i in range(_L)] + [kv1[i] for i in range(_L)]

        def col_body(jj, c):
            col = jj * (2 * _L)
            for u in range(2):
                cu = col + u * _L
                p = [None] * 4
                for k in range(_K):
                    term = vals[k] * rows_v[k, pl.ds(cu, _L)]
                    q = k & 3
                    p[q] = term if p[q] is None else p[q] + term
                outv[pl.ds(cu, _L)] = (p[0] + p[1]) + (p[2] + p[3]) + pbv[pl.ds(cu, _L)]
            return c

        lax.fori_loop(0, _INPUT_DIM // (2 * _L), col_body, 0)

    with jax.named_scope("sc_dma_out"):
        pltpu.sync_copy(outv, out_hbm.at[wid])
        cp_sparse.wait()


def _sc_topk_decode(latents, gmax, w, pre_bias):
    latf = latents.reshape(_BATCH * _NGRP, _GSZ)
    mesh = plsc.VectorSubcoreMesh(
        core_axis_name="c", subcore_axis_name="s",
        num_cores=_NC, num_subcores=_NS,
    )
    f = pl.kernel(
        _sc_body,
        out_type=[
            jax.ShapeDtypeStruct((_BATCH, _INPUT_DIM), jnp.float32),
            jax.ShapeDtypeStruct((_BATCH, _LATENT_DIM), jnp.float32),
        ],
        mesh=mesh,
        compiler_params=pltpu.CompilerParams(needs_layout_passes=False),
        scratch_types=[
            pltpu.VMEM((_LATENT_DIM,), jnp.float32),
            pltpu.VMEM((_K, _GSZ), jnp.float32),
            pltpu.VMEM((_K, _INPUT_DIM), jnp.float32),
            pltpu.VMEM((_INPUT_DIM,), jnp.float32),
            pltpu.VMEM((_INPUT_DIM,), jnp.float32),
            pltpu.VMEM((_NGRP,), jnp.float32),
            pltpu.VMEM((_K * _GSZ + _L,), jnp.float32),
            pltpu.VMEM((_K * _GSZ + _L,), jnp.int32),
            pltpu.VMEM((2 * _L,), jnp.int32),
            pltpu.VMEM((2 * _L,), jnp.int32),
            pltpu.SemaphoreType.DMA,
            pltpu.SemaphoreType.DMA,
            pltpu.SemaphoreType.DMA,
            pltpu.SemaphoreType.DMA,
        ],
    )
    return f(latf, gmax, w, pre_bias)


@jax.jit
def kernel(x, W_enc, b_enc, pre_bias):
    b2 = b_enc.reshape(_NBLK, 1, _LBLK)
    latents, gmax4 = _encode(x, W_enc, b2)
    gmax = gmax4.reshape(_BATCH, _NGRP)
    output, sparse_latents = _sc_topk_decode(latents, gmax, W_enc, pre_bias)
    return (output, sparse_latents)


# speedup vs baseline: 2.2967x; 1.0126x over previous
import jax
import jax.numpy as jnp
from jax import lax
from jax.experimental import pallas as pl
from jax.experimental.pallas import tpu as pltpu
from jax.experimental.pallas import tpu_sc as plsc

_INPUT_DIM = 2048
_LATENT_DIM = 32768
_BATCH = 32
_K = 32
_LBLK = 2048
_NBLK = _LATENT_DIM // _LBLK

_NC = 2
_NS = 16
_L = 16

_GSZ = 128
_NGRP = _LATENT_DIM // _GSZ
_GPB = _LBLK // _GSZ

_NEG_INF = float("-inf")


def _enc_body(x_ref, w_ref, b_ref, lat_ref, gm_ref, xn_ref):
    j = pl.program_id(0)

    @pl.when(j == 0)
    def _():
        x = x_ref[...]
        mu = jnp.mean(x, axis=1, keepdims=True)
        xc = x - mu
        var = jnp.sum(xc * xc, axis=1, keepdims=True) / (_INPUT_DIM - 1)
        std = jnp.sqrt(var)
        xn_ref[...] = xc / (std + 1e-5)

    xn = xn_ref[...]
    w = w_ref[...]
    acc = lax.dot_general(
        xn, w, (((1,), (1,)), ((), ())),
        preferred_element_type=jnp.float32,
    )
    lat = acc + b_ref[0]
    lat_ref[...] = lat
    gm = jnp.max(lat.reshape(_BATCH, _GPB, _GSZ), axis=2)
    gm_ref[...] = gm.reshape(_BATCH, 1, 1, _GPB)


def _encode(x, w, b2):
    return pl.pallas_call(
        _enc_body,
        grid=(_NBLK,),
        in_specs=[
            pl.BlockSpec((_BATCH, _INPUT_DIM), lambda j: (0, 0)),
            pl.BlockSpec((_LBLK, _INPUT_DIM), lambda j: (j, 0)),
            pl.BlockSpec((1, 1, _LBLK), lambda j: (j, 0, 0)),
        ],
        out_specs=[
            pl.BlockSpec((_BATCH, _LBLK), lambda j: (0, j)),
            pl.BlockSpec((_BATCH, 1, 1, _GPB), lambda j: (0, j, 0, 0)),
        ],
        out_shape=[
            jax.ShapeDtypeStruct((_BATCH, _LATENT_DIM), jnp.float32),
            jax.ShapeDtypeStruct((_BATCH, _NBLK, 1, _GPB), jnp.float32),
        ],
        scratch_shapes=[pltpu.VMEM((_BATCH, _INPUT_DIM), jnp.float32)],
        compiler_params=pltpu.CompilerParams(vmem_limit_bytes=63 * 1024 * 1024),
    )(x, w, b2)


def _merge_topk(kv0, kv1, ki0, ki1, sv, si):
    rsv = lax.rev(sv, (0,))
    rsi = lax.rev(si, (0,))
    ge = kv1 >= rsv
    c1 = jnp.where(ge, kv1, rsv)
    c1i = jnp.where(ge, ki1, rsi)
    ge2 = kv0 >= c1
    hi = jnp.where(ge2, kv0, c1)
    hii = jnp.where(ge2, ki0, c1i)
    lo = jnp.where(ge2, c1, kv0)
    loi = jnp.where(ge2, c1i, ki0)
    kv0, ki0 = plsc.sort_key_val(hi, hii, descending=True)
    kv1, ki1 = plsc.sort_key_val(lo, loi, descending=True)
    return kv0, kv1, ki0, ki1


def _sc_body(latf_hbm, gmax_hbm, w_hbm, pb_hbm, out_hbm, sparse_hbm,
             zbuf, grp_rows, rows_v, outv, pbv, gmaxv,
             cand_v, cand_i, ki_ref, gi_ref, sem0, sem1, sem2, sem3):
    wid = lax.axis_index("s") * _NC + lax.axis_index("c")

    lane = lax.iota(jnp.int32, _L)
    zeros16 = jnp.zeros((_L,), jnp.float32)
    neg_inf16 = jnp.full((_L,), _NEG_INF, jnp.float32)
    izeros16 = jnp.zeros((_L,), jnp.int32)

    with jax.named_scope("sc_dma_in"):
        cp_gm = pltpu.make_async_copy(gmax_hbm.at[wid], gmaxv, sem0)
        cp_gm.start()
        cp_pb = pltpu.make_async_copy(pb_hbm, pbv, sem1)
        cp_pb.start()

    with jax.named_scope("sc_phase_a"):
        cp_gm.wait()
        gv0, gv1 = neg_inf16, neg_inf16
        gi0, gi1 = izeros16, izeros16
        for c in range(_NGRP // _L):
            v = gmaxv[pl.ds(c * _L, _L)]
            sv, si = plsc.sort_key_val(v, c * _L + lane, descending=True)
            gv0, gv1, gi0, gi1 = _merge_topk(gv0, gv1, gi0, gi1, sv, si)
        gi_ref[pl.ds(0, _L)] = gi0 + wid * _NGRP
        gi_ref[pl.ds(_L, _L)] = gi1 + wid * _NGRP
        g_scal = [gi0[i] for i in range(_L)] + [gi1[i] for i in range(_L)]
        t0g = gv1[_L - 1]

    with jax.named_scope("sc_group_gather"):
        cp_grp = pltpu.make_async_copy(latf_hbm.at[gi_ref], grp_rows, sem2)
        cp_grp.start()
        cp_grp.wait()

    with jax.named_scope("sc_phase_b"):
        cnt = jnp.int32(0)
        for k in range(_K):
            g = g_scal[k]

            def chunk_body(c, cnt, k=k, g=g):
                v = grp_rows[k, pl.ds(c * _L, _L)]
                for u in range(_GSZ // _L):
                    zbuf[pl.ds(k * 1024 + c * _GSZ + u * _L, _L)] = zeros16
                m = v >= t0g
                iv = g * _GSZ + c * _L + lane
                plsc.store_compressed(cand_v.at[pl.ds(cnt, _L)], v, mask=m)
                plsc.store_compressed(cand_i.at[pl.ds(cnt, _L)], iv, mask=m)
                return cnt + plsc.all_reduce_population_count(m)[0]

            cnt = lax.fori_loop(0, _GSZ // _L, chunk_body, cnt)

        def p2_body(i, carry):
            kv0, kv1, ki0, ki1 = carry
            v = cand_v[pl.ds(i * _L, _L)]
            iv = cand_i[pl.ds(i * _L, _L)]
            valid = (i * _L + lane) < cnt
            v = jnp.where(valid, v, neg_inf16)
            sv, si = plsc.sort_key_val(v, iv, descending=True)
            return _merge_topk(kv0, kv1, ki0, ki1, sv, si)

        nv = (cnt + _L - 1) // _L
        kv0, kv1, ki0, ki1 = lax.fori_loop(
            0, nv, p2_body, (neg_inf16, neg_inf16, izeros16, izeros16))

    with jax.named_scope("sc_scatter"):
        ki_ref[pl.ds(0, _L)] = ki0
        ki_ref[pl.ds(_L, _L)] = ki1
        plsc.store_scatter(zbuf, [ki0], kv0)
        plsc.store_scatter(zbuf, [ki1], kv1)

        cp_w = pltpu.make_async_copy(w_hbm.at[ki_ref], rows_v, sem3)
        cp_w.start()

    with jax.named_scope("sc_gather_wait"):
        cp_pb.wait()
        cp_w.wait()
        cp_sparse = pltpu.make_async_copy(zbuf, sparse_hbm.at[wid], sem0)
        cp_sparse.start()

    with jax.named_scope("sc_decode"):
        vals = [kv0[i] for i in range(_L)] + [kv1[i] for i in range(_L)]

        def col_body(jj, c):
            col = jj * (2 * _L)
            for u in range(2):
                cu = col + u * _L
                p = [None] * 4
                for k in range(_K):
                    term = vals[k] * rows_v[k, pl.ds(cu, _L)]
                    q = k & 3
                    p[q] = term if p[q] is None else p[q] + term
                outv[pl.ds(cu, _L)] = (p[0] + p[1]) + (p[2] + p[3]) + pbv[pl.ds(cu, _L)]
            return c

        lax.fori_loop(0, _INPUT_DIM // (2 * _L), col_body, 0)

    with jax.named_scope("sc_dma_out"):
        pltpu.sync_copy(outv, out_hbm.at[wid])
        cp_sparse.wait()


def _sc_topk_decode(latents, gmax, w, pre_bias):
    latf = latents.reshape(_BATCH * _NGRP, _GSZ)
    mesh = plsc.VectorSubcoreMesh(
        core_axis_name="c", subcore_axis_name="s",
        num_cores=_NC, num_subcores=_NS,
    )
    f = pl.kernel(
        _sc_body,
        out_type=[
            jax.ShapeDtypeStruct((_BATCH, _INPUT_DIM), jnp.float32),
            jax.ShapeDtypeStruct((_BATCH, _LATENT_DIM), jnp.float32),
        ],
        mesh=mesh,
        compiler_params=pltpu.CompilerParams(needs_layout_passes=False),
        scratch_types=[
            pltpu.VMEM((_LATENT_DIM,), jnp.float32),
            pltpu.VMEM((_K, _GSZ), jnp.float32),
            pltpu.VMEM((_K, _INPUT_DIM), jnp.float32),
            pltpu.VMEM((_INPUT_DIM,), jnp.float32),
            pltpu.VMEM((_INPUT_DIM,), jnp.float32),
            pltpu.VMEM((_NGRP,), jnp.float32),
            pltpu.VMEM((_K * _GSZ + _L,), jnp.float32),
            pltpu.VMEM((_K * _GSZ + _L,), jnp.int32),
            pltpu.VMEM((2 * _L,), jnp.int32),
            pltpu.VMEM((2 * _L,), jnp.int32),
            pltpu.SemaphoreType.DMA,
            pltpu.SemaphoreType.DMA,
            pltpu.SemaphoreType.DMA,
            pltpu.SemaphoreType.DMA,
        ],
    )
    return f(latf, gmax, w, pre_bias)


@jax.jit
def kernel(x, W_enc, b_enc, pre_bias):
    b2 = b_enc.reshape(_NBLK, 1, _LBLK)
    latents, gmax4 = _encode(x, W_enc, b2)
    gmax = gmax4.reshape(_BATCH, _NGRP)
    output, sparse_latents = _sc_topk_decode(latents, gmax, W_enc, pre_bias)
    return (output, sparse_latents)


# speedup vs baseline: 2.3996x; 1.0448x over previous
import jax
import jax.numpy as jnp
from jax import lax
from jax.experimental import pallas as pl
from jax.experimental.pallas import tpu as pltpu
from jax.experimental.pallas import tpu_sc as plsc

_INPUT_DIM = 2048
_LATENT_DIM = 32768
_BATCH = 32
_K = 32
_LBLK = 2048
_NBLK = _LATENT_DIM // _LBLK

_NC = 2
_NS = 16
_L = 16

_GSZ = 128
_NGRP = _LATENT_DIM // _GSZ
_GPB = _LBLK // _GSZ

_NEG_INF = float("-inf")


def _enc_body(x_ref, w_ref, b_ref, lat_ref, gm_ref, xn_ref):
    j = pl.program_id(0)

    @pl.when(j == 0)
    def _():
        x = x_ref[...]
        mu = jnp.mean(x, axis=1, keepdims=True)
        xc = x - mu
        var = jnp.sum(xc * xc, axis=1, keepdims=True) / (_INPUT_DIM - 1)
        std = jnp.sqrt(var)
        xn_ref[...] = xc / (std + 1e-5)

    xn = xn_ref[...]
    w = w_ref[...]
    acc = lax.dot_general(
        xn, w, (((1,), (1,)), ((), ())),
        preferred_element_type=jnp.float32,
    )
    lat = acc + b_ref[0]
    lat3 = lat.reshape(_BATCH, _GPB, _GSZ)
    lat_ref[...] = lat3
    gm = jnp.max(lat3, axis=2)
    gm_ref[...] = gm.reshape(_BATCH, 1, 1, _GPB)


def _encode(x, w, b2):
    return pl.pallas_call(
        _enc_body,
        grid=(_NBLK,),
        in_specs=[
            pl.BlockSpec((_BATCH, _INPUT_DIM), lambda j: (0, 0)),
            pl.BlockSpec((_LBLK, _INPUT_DIM), lambda j: (j, 0)),
            pl.BlockSpec((1, 1, _LBLK), lambda j: (j, 0, 0)),
        ],
        out_specs=[
            pl.BlockSpec((_BATCH, _GPB, _GSZ), lambda j: (0, j, 0)),
            pl.BlockSpec((_BATCH, 1, 1, _GPB), lambda j: (0, j, 0, 0)),
        ],
        out_shape=[
            jax.ShapeDtypeStruct((_BATCH, _NGRP, _GSZ), jnp.float32),
            jax.ShapeDtypeStruct((_BATCH, _NBLK, 1, _GPB), jnp.float32),
        ],
        scratch_shapes=[pltpu.VMEM((_BATCH, _INPUT_DIM), jnp.float32)],
        compiler_params=pltpu.CompilerParams(vmem_limit_bytes=63 * 1024 * 1024),
    )(x, w, b2)


def _merge_topk(kv0, kv1, ki0, ki1, sv, si):
    rsv = lax.rev(sv, (0,))
    rsi = lax.rev(si, (0,))
    ge = kv1 >= rsv
    c1 = jnp.where(ge, kv1, rsv)
    c1i = jnp.where(ge, ki1, rsi)
    ge2 = kv0 >= c1
    hi = jnp.where(ge2, kv0, c1)
    hii = jnp.where(ge2, ki0, c1i)
    lo = jnp.where(ge2, c1, kv0)
    loi = jnp.where(ge2, c1i, ki0)
    kv0, ki0 = plsc.sort_key_val(hi, hii, descending=True)
    kv1, ki1 = plsc.sort_key_val(lo, loi, descending=True)
    return kv0, kv1, ki0, ki1


_Z1 = 512


def _sc_body(lat3_hbm, gmax_hbm, w_hbm, pb_hbm, out_hbm, sparse_hbm,
             zbuf, grp_rows, rows_v, outv, pbv, gmaxv,
             cand_v, cand_i, ki_ref, gi_ref, sem0, sem1, sem2, sem3):
    wid = lax.axis_index("s") * _NC + lax.axis_index("c")

    lane = lax.iota(jnp.int32, _L)
    zeros16 = jnp.zeros((_L,), jnp.float32)
    neg_inf16 = jnp.full((_L,), _NEG_INF, jnp.float32)
    izeros16 = jnp.zeros((_L,), jnp.int32)

    with jax.named_scope("sc_dma_in"):
        cp_gm = pltpu.make_async_copy(gmax_hbm.at[wid], gmaxv, sem0)
        cp_gm.start()
        cp_pb = pltpu.make_async_copy(pb_hbm, pbv, sem1)
        cp_pb.start()

    with jax.named_scope("sc_phase_a"):
        cp_gm.wait()
        gv0, gv1 = neg_inf16, neg_inf16
        gi0, gi1 = izeros16, izeros16
        for c in range(_NGRP // _L):
            v = gmaxv[pl.ds(c * _L, _L)]
            sv, si = plsc.sort_key_val(v, c * _L + lane, descending=True)
            gv0, gv1, gi0, gi1 = _merge_topk(gv0, gv1, gi0, gi1, sv, si)
        gi_ref[pl.ds(0, _L)] = gi0
        gi_ref[pl.ds(_L, _L)] = gi1
        g_scal = [gi0[i] for i in range(_L)] + [gi1[i] for i in range(_L)]
        t0g = gv1[_L - 1]

    with jax.named_scope("sc_group_gather"):
        cp_grp = pltpu.make_async_copy(lat3_hbm.at[wid].at[gi_ref], grp_rows, sem2)
        cp_grp.start()
        def zero1_body(i, c):
            for u in range(4):
                zbuf[pl.ds(i * (4 * _L) + u * _L, _L)] = zeros16
            return c

        lax.fori_loop(0, _Z1 // 4, zero1_body, 0)
        cp_grp.wait()

    with jax.named_scope("sc_phase_b"):
        cnt = jnp.int32(0)
        for k in range(_K):
            g = g_scal[k]

            def chunk_body(c, cnt, k=k, g=g):
                v = grp_rows[k, pl.ds(c * _L, _L)]
                m = v >= t0g
                iv = g * _GSZ + c * _L + lane
                plsc.store_compressed(cand_v.at[pl.ds(cnt, _L)], v, mask=m)
                plsc.store_compressed(cand_i.at[pl.ds(cnt, _L)], iv, mask=m)
                return cnt + plsc.all_reduce_population_count(m)[0]

            cnt = lax.fori_loop(0, _GSZ // _L, chunk_body, cnt)

        def p2_body(i, carry):
            kv0, kv1, ki0, ki1 = carry
            v = cand_v[pl.ds(i * _L, _L)]
            iv = cand_i[pl.ds(i * _L, _L)]
            valid = (i * _L + lane) < cnt
            v = jnp.where(valid, v, neg_inf16)
            sv, si = plsc.sort_key_val(v, iv, descending=True)
            return _merge_topk(kv0, kv1, ki0, ki1, sv, si)

        nv = (cnt + _L - 1) // _L
        kv0, kv1, ki0, ki1 = lax.fori_loop(
            0, nv, p2_body, (neg_inf16, neg_inf16, izeros16, izeros16))

    with jax.named_scope("sc_scatter"):
        ki_ref[pl.ds(0, _L)] = ki0
        ki_ref[pl.ds(_L, _L)] = ki1

        cp_w = pltpu.make_async_copy(w_hbm.at[ki_ref], rows_v, sem3)
        cp_w.start()

        def zero2_body(i, c):
            base = _Z1 * _L + i * (4 * _L)
            for u in range(4):
                zbuf[pl.ds(base + u * _L, _L)] = zeros16
            return c

        lax.fori_loop(0, (_LATENT_DIM // _L - _Z1) // 4, zero2_body, 0)
        plsc.store_scatter(zbuf, [ki0], kv0)
        plsc.store_scatter(zbuf, [ki1], kv1)

    with jax.named_scope("sc_gather_wait"):
        cp_pb.wait()
        cp_w.wait()
        cp_sparse = pltpu.make_async_copy(zbuf, sparse_hbm.at[wid], sem0)
        cp_sparse.start()

    with jax.named_scope("sc_decode"):
        vals = [kv0[i] for i in range(_L)] + [kv1[i] for i in range(_L)]

        def col_body(jj, c):
            col = jj * (2 * _L)
            for u in range(2):
                cu = col + u * _L
                p = [None] * 4
                for k in range(_K):
                    term = vals[k] * rows_v[k, pl.ds(cu, _L)]
                    q = k & 3
                    p[q] = term if p[q] is None else p[q] + term
                outv[pl.ds(cu, _L)] = (p[0] + p[1]) + (p[2] + p[3]) + pbv[pl.ds(cu, _L)]
            return c

        lax.fori_loop(0, _INPUT_DIM // (2 * _L), col_body, 0)

    with jax.named_scope("sc_dma_out"):
        pltpu.sync_copy(outv, out_hbm.at[wid])
        cp_sparse.wait()


def _sc_topk_decode(lat3, gmax, w, pre_bias):
    mesh = plsc.VectorSubcoreMesh(
        core_axis_name="c", subcore_axis_name="s",
        num_cores=_NC, num_subcores=_NS,
    )
    f = pl.kernel(
        _sc_body,
        out_type=[
            jax.ShapeDtypeStruct((_BATCH, _INPUT_DIM), jnp.float32),
            jax.ShapeDtypeStruct((_BATCH, _LATENT_DIM), jnp.float32),
        ],
        mesh=mesh,
        compiler_params=pltpu.CompilerParams(needs_layout_passes=False),
        scratch_types=[
            pltpu.VMEM((_LATENT_DIM,), jnp.float32),
            pltpu.VMEM((_K, _GSZ), jnp.float32),
            pltpu.VMEM((_K, _INPUT_DIM), jnp.float32),
            pltpu.VMEM((_INPUT_DIM,), jnp.float32),
            pltpu.VMEM((_INPUT_DIM,), jnp.float32),
            pltpu.VMEM((_NGRP,), jnp.float32),
            pltpu.VMEM((_K * _GSZ + _L,), jnp.float32),
            pltpu.VMEM((_K * _GSZ + _L,), jnp.int32),
            pltpu.VMEM((2 * _L,), jnp.int32),
            pltpu.VMEM((2 * _L,), jnp.int32),
            pltpu.SemaphoreType.DMA,
            pltpu.SemaphoreType.DMA,
            pltpu.SemaphoreType.DMA,
            pltpu.SemaphoreType.DMA,
        ],
    )
    return f(lat3, gmax, w, pre_bias)


@jax.jit
def kernel(x, W_enc, b_enc, pre_bias):
    b2 = b_enc.reshape(_NBLK, 1, _LBLK)
    lat3, gmax4 = _encode(x, W_enc, b2)
    gmax = gmax4.reshape(_BATCH, _NGRP)
    output, sparse_latents = _sc_topk_decode(lat3, gmax, W_enc, pre_bias)
    return (output, sparse_latents)


# speedup vs baseline: 2.4247x; 1.0104x over previous
import jax
import jax.numpy as jnp
from jax import lax
from jax.experimental import pallas as pl
from jax.experimental.pallas import tpu as pltpu
from jax.experimental.pallas import tpu_sc as plsc

_INPUT_DIM = 2048
_LATENT_DIM = 32768
_BATCH = 32
_K = 32
_LBLK = 2048
_NBLK = _LATENT_DIM // _LBLK

_NC = 2
_NS = 16
_L = 16

_GSZ = 128
_NGRP = _LATENT_DIM // _GSZ
_GPB = _LBLK // _GSZ

_NEG_INF = float("-inf")


def _enc_body(x_ref, w_ref, b_ref, lat_ref, gm_ref, xn_ref):
    j = pl.program_id(0)

    @pl.when(j == 0)
    def _():
        x = x_ref[...]
        mu = jnp.mean(x, axis=1, keepdims=True)
        xc = x - mu
        var = jnp.sum(xc * xc, axis=1, keepdims=True) / (_INPUT_DIM - 1)
        std = jnp.sqrt(var)
        xn_ref[...] = xc / (std + 1e-5)

    xn = xn_ref[...]
    w = w_ref[...]
    acc = lax.dot_general(
        xn, w, (((1,), (1,)), ((), ())),
        preferred_element_type=jnp.float32,
    )
    lat = acc + b_ref[0]
    lat3 = lat.reshape(_BATCH, _GPB, _GSZ)
    lat_ref[...] = lat3
    gm = jnp.max(lat3, axis=2)
    gm_ref[...] = gm.reshape(_BATCH, 1, 1, _GPB)


def _encode(x, w, b2):
    return pl.pallas_call(
        _enc_body,
        grid=(_NBLK,),
        in_specs=[
            pl.BlockSpec((_BATCH, _INPUT_DIM), lambda j: (0, 0)),
            pl.BlockSpec((_LBLK, _INPUT_DIM), lambda j: (j, 0)),
            pl.BlockSpec((1, 1, _LBLK), lambda j: (j, 0, 0)),
        ],
        out_specs=[
            pl.BlockSpec((_BATCH, _GPB, _GSZ), lambda j: (0, j, 0)),
            pl.BlockSpec((_BATCH, 1, 1, _GPB), lambda j: (0, j, 0, 0)),
        ],
        out_shape=[
            jax.ShapeDtypeStruct((_BATCH, _NGRP, _GSZ), jnp.float32),
            jax.ShapeDtypeStruct((_BATCH, _NBLK, 1, _GPB), jnp.float32),
        ],
        scratch_shapes=[pltpu.VMEM((_BATCH, _INPUT_DIM), jnp.float32)],
        compiler_params=pltpu.CompilerParams(vmem_limit_bytes=63 * 1024 * 1024),
    )(x, w, b2)


def _merge_topk(kv0, kv1, ki0, ki1, sv, si):
    rsv = lax.rev(sv, (0,))
    rsi = lax.rev(si, (0,))
    ge = kv1 >= rsv
    c1 = jnp.where(ge, kv1, rsv)
    c1i = jnp.where(ge, ki1, rsi)
    ge2 = kv0 >= c1
    hi = jnp.where(ge2, kv0, c1)
    hii = jnp.where(ge2, ki0, c1i)
    lo = jnp.where(ge2, c1, kv0)
    loi = jnp.where(ge2, c1i, ki0)
    kv0, ki0 = plsc.sort_key_val(hi, hii, descending=True)
    kv1, ki1 = plsc.sort_key_val(lo, loi, descending=True)
    return kv0, kv1, ki0, ki1


_Z1 = 512


def _sc_body(lat3_hbm, gmax_hbm, w_hbm, pb_hbm, out_hbm, sparse_hbm,
             zbuf, grp_rows, rows_v, outv, pbv, gmaxv,
             cand_v, cand_i, ki_ref, gi_ref, sem0, sem1, sem2, sem3):
    wid = lax.axis_index("s") * _NC + lax.axis_index("c")

    lane = lax.iota(jnp.int32, _L)
    zeros16 = jnp.zeros((_L,), jnp.float32)
    neg_inf16 = jnp.full((_L,), _NEG_INF, jnp.float32)
    izeros16 = jnp.zeros((_L,), jnp.int32)

    with jax.named_scope("sc_dma_in"):
        cp_gm = pltpu.make_async_copy(gmax_hbm.at[wid], gmaxv, sem0)
        cp_gm.start()
        cp_pb = pltpu.make_async_copy(pb_hbm, pbv, sem1)
        cp_pb.start()

    with jax.named_scope("sc_phase_a"):
        cp_gm.wait()
        gv0, gv1 = neg_inf16, neg_inf16
        gi0, gi1 = izeros16, izeros16
        for c in range(_NGRP // _L):
            v = gmaxv[c, 0, :]
            sv, si = plsc.sort_key_val(v, c * _L + lane, descending=True)
            gv0, gv1, gi0, gi1 = _merge_topk(gv0, gv1, gi0, gi1, sv, si)
        gi_ref[pl.ds(0, _L)] = gi0
        gi_ref[pl.ds(_L, _L)] = gi1
        g_scal = [gi0[i] for i in range(_L)] + [gi1[i] for i in range(_L)]
        t0g = gv1[_L - 1]

    with jax.named_scope("sc_group_gather"):
        cp_grp = pltpu.make_async_copy(lat3_hbm.at[wid].at[gi_ref], grp_rows, sem2)
        cp_grp.start()
        def zero1_body(i, c):
            for u in range(4):
                zbuf[pl.ds(i * (4 * _L) + u * _L, _L)] = zeros16
            return c

        lax.fori_loop(0, _Z1 // 4, zero1_body, 0)
        cp_grp.wait()

    with jax.named_scope("sc_phase_b"):
        cnt = jnp.int32(0)
        for k in range(_K):
            g = g_scal[k]

            def chunk_body(c, cnt, k=k, g=g):
                v = grp_rows[k, pl.ds(c * _L, _L)]
                m = v >= t0g
                iv = g * _GSZ + c * _L + lane
                plsc.store_compressed(cand_v.at[pl.ds(cnt, _L)], v, mask=m)
                plsc.store_compressed(cand_i.at[pl.ds(cnt, _L)], iv, mask=m)
                return cnt + plsc.all_reduce_population_count(m)[0]

            cnt = lax.fori_loop(0, _GSZ // _L, chunk_body, cnt)

        def p2_body(i, carry):
            kv0, kv1, ki0, ki1 = carry
            v = cand_v[pl.ds(i * _L, _L)]
            iv = cand_i[pl.ds(i * _L, _L)]
            valid = (i * _L + lane) < cnt
            v = jnp.where(valid, v, neg_inf16)
            sv, si = plsc.sort_key_val(v, iv, descending=True)
            return _merge_topk(kv0, kv1, ki0, ki1, sv, si)

        nv = (cnt + _L - 1) // _L
        kv0, kv1, ki0, ki1 = lax.fori_loop(
            0, nv, p2_body, (neg_inf16, neg_inf16, izeros16, izeros16))

    with jax.named_scope("sc_scatter"):
        ki_ref[pl.ds(0, _L)] = ki0
        ki_ref[pl.ds(_L, _L)] = ki1

        cp_w = pltpu.make_async_copy(w_hbm.at[ki_ref], rows_v, sem3)
        cp_w.start()

        def zero2_body(i, c):
            base = _Z1 * _L + i * (4 * _L)
            for u in range(4):
                zbuf[pl.ds(base + u * _L, _L)] = zeros16
            return c

        lax.fori_loop(0, (_LATENT_DIM // _L - _Z1) // 4, zero2_body, 0)
        plsc.store_scatter(zbuf, [ki0], kv0)
        plsc.store_scatter(zbuf, [ki1], kv1)

    with jax.named_scope("sc_gather_wait"):
        cp_pb.wait()
        cp_w.wait()
        cp_sparse = pltpu.make_async_copy(zbuf, sparse_hbm.at[wid], sem0)
        cp_sparse.start()

    with jax.named_scope("sc_decode"):
        vals = [kv0[i] for i in range(_L)] + [kv1[i] for i in range(_L)]

        def col_body(jj, c):
            col = jj * (2 * _L)
            for u in range(2):
                cu = col + u * _L
                p = [None] * 4
                for k in range(_K):
                    term = vals[k] * rows_v[k, pl.ds(cu, _L)]
                    q = k & 3
                    p[q] = term if p[q] is None else p[q] + term
                outv[pl.ds(cu, _L)] = (p[0] + p[1]) + (p[2] + p[3]) + pbv[pl.ds(cu, _L)]
            return c

        lax.fori_loop(0, _INPUT_DIM // (2 * _L), col_body, 0)

    with jax.named_scope("sc_dma_out"):
        pltpu.sync_copy(outv, out_hbm.at[wid])
        cp_sparse.wait()


def _sc_topk_decode(lat3, gmax, w, pre_bias):
    mesh = plsc.VectorSubcoreMesh(
        core_axis_name="c", subcore_axis_name="s",
        num_cores=_NC, num_subcores=_NS,
    )
    f = pl.kernel(
        _sc_body,
        out_type=[
            jax.ShapeDtypeStruct((_BATCH, _INPUT_DIM), jnp.float32),
            jax.ShapeDtypeStruct((_BATCH, _LATENT_DIM), jnp.float32),
        ],
        mesh=mesh,
        compiler_params=pltpu.CompilerParams(needs_layout_passes=False),
        scratch_types=[
            pltpu.VMEM((_LATENT_DIM,), jnp.float32),
            pltpu.VMEM((_K, _GSZ), jnp.float32),
            pltpu.VMEM((_K, _INPUT_DIM), jnp.float32),
            pltpu.VMEM((_INPUT_DIM,), jnp.float32),
            pltpu.VMEM((_INPUT_DIM,), jnp.float32),
            pltpu.VMEM((_NBLK, 1, _GPB), jnp.float32),
            pltpu.VMEM((_K * _GSZ + _L,), jnp.float32),
            pltpu.VMEM((_K * _GSZ + _L,), jnp.int32),
            pltpu.VMEM((2 * _L,), jnp.int32),
            pltpu.VMEM((2 * _L,), jnp.int32),
            pltpu.SemaphoreType.DMA,
            pltpu.SemaphoreType.DMA,
            pltpu.SemaphoreType.DMA,
            pltpu.SemaphoreType.DMA,
        ],
    )
    return f(lat3, gmax, w, pre_bias)


@jax.jit
def kernel(x, W_enc, b_enc, pre_bias):
    b2 = b_enc.reshape(_NBLK, 1, _LBLK)
    lat3, gmax4 = _encode(x, W_enc, b2)
    output, sparse_latents = _sc_topk_decode(lat3, gmax4, W_enc, pre_bias)
    return (output, sparse_latents)


# speedup vs baseline: 2.4702x; 1.0188x over previous
import jax
import jax.numpy as jnp
from jax import lax
from jax.experimental import pallas as pl
from jax.experimental.pallas import tpu as pltpu
from jax.experimental.pallas import tpu_sc as plsc

_INPUT_DIM = 2048
_LATENT_DIM = 32768
_BATCH = 32
_K = 32
_LBLK = 2048
_NBLK = _LATENT_DIM // _LBLK

_NC = 2
_NS = 16
_L = 16

_GSZ = 128
_NGRP = _LATENT_DIM // _GSZ
_GPB = _LBLK // _GSZ

_NEG_INF = float("-inf")


_HBLK = _LBLK // 2


def _enc_body(x_ref, wa_ref, wb_ref, b_ref, lat_ref, gm_ref, xn_ref):
    j = pl.program_id(0)

    @pl.when(j == 0)
    def _():
        x = x_ref[...]
        mu = jnp.mean(x, axis=1, keepdims=True)
        xc = x - mu
        var = jnp.sum(xc * xc, axis=1, keepdims=True) / (_INPUT_DIM - 1)
        std = jnp.sqrt(var)
        xn_ref[...] = xc / (std + 1e-5)

    xn = xn_ref[...]
    for h, w_ref in enumerate((wa_ref, wb_ref)):
        acc = lax.dot_general(
            xn, w_ref[...], (((1,), (1,)), ((), ())),
            preferred_element_type=jnp.float32,
        )
        lat = acc + b_ref[0, :, pl.ds(h * _HBLK, _HBLK)][0]
        lat3 = lat.reshape(_BATCH, _GPB // 2, _GSZ)
        lat_ref[:, pl.ds(h * (_GPB // 2), _GPB // 2), :] = lat3
        gm = jnp.max(lat3, axis=2)
        gm_ref[:, 0, 0, pl.ds(h * (_GPB // 2), _GPB // 2)] = gm


def _encode(x, w, b2):
    return pl.pallas_call(
        _enc_body,
        grid=(_NBLK,),
        in_specs=[
            pl.BlockSpec((_BATCH, _INPUT_DIM), lambda j: (0, 0)),
            pl.BlockSpec((_HBLK, _INPUT_DIM), lambda j: (2 * j, 0)),
            pl.BlockSpec((_HBLK, _INPUT_DIM), lambda j: (2 * j + 1, 0)),
            pl.BlockSpec((1, 1, _LBLK), lambda j: (j, 0, 0)),
        ],
        out_specs=[
            pl.BlockSpec((_BATCH, _GPB, _GSZ), lambda j: (0, j, 0)),
            pl.BlockSpec((_BATCH, 1, 1, _GPB), lambda j: (0, j, 0, 0)),
        ],
        out_shape=[
            jax.ShapeDtypeStruct((_BATCH, _NGRP, _GSZ), jnp.float32),
            jax.ShapeDtypeStruct((_BATCH, _NBLK, 1, _GPB), jnp.float32),
        ],
        scratch_shapes=[pltpu.VMEM((_BATCH, _INPUT_DIM), jnp.float32)],
        compiler_params=pltpu.CompilerParams(vmem_limit_bytes=63 * 1024 * 1024),
    )(x, w, w, b2)


def _merge_topk(kv0, kv1, ki0, ki1, sv, si):
    rsv = lax.rev(sv, (0,))
    rsi = lax.rev(si, (0,))
    ge = kv1 >= rsv
    c1 = jnp.where(ge, kv1, rsv)
    c1i = jnp.where(ge, ki1, rsi)
    ge2 = kv0 >= c1
    hi = jnp.where(ge2, kv0, c1)
    hii = jnp.where(ge2, ki0, c1i)
    lo = jnp.where(ge2, c1, kv0)
    loi = jnp.where(ge2, c1i, ki0)
    kv0, ki0 = plsc.sort_key_val(hi, hii, descending=True)
    kv1, ki1 = plsc.sort_key_val(lo, loi, descending=True)
    return kv0, kv1, ki0, ki1


_Z1 = 512


def _sc_body(lat3_hbm, gmax_hbm, w_hbm, pb_hbm, out_hbm, sparse_hbm,
             zbuf, grp_rows, rows_v, outv, pbv, gmaxv,
             cand_v, cand_i, ki_ref, gi_ref, sem0, sem1, sem2, sem3):
    wid = lax.axis_index("s") * _NC + lax.axis_index("c")

    lane = lax.iota(jnp.int32, _L)
    zeros16 = jnp.zeros((_L,), jnp.float32)
    neg_inf16 = jnp.full((_L,), _NEG_INF, jnp.float32)
    izeros16 = jnp.zeros((_L,), jnp.int32)

    with jax.named_scope("sc_dma_in"):
        cp_gm = pltpu.make_async_copy(gmax_hbm.at[wid], gmaxv, sem0)
        cp_gm.start()
        cp_pb = pltpu.make_async_copy(pb_hbm, pbv, sem1)
        cp_pb.start()

    with jax.named_scope("sc_phase_a"):
        cp_gm.wait()
        gv0, gv1 = neg_inf16, neg_inf16
        gi0, gi1 = izeros16, izeros16
        for c in range(_NGRP // _L):
            v = gmaxv[c, 0, :]
            sv, si = plsc.sort_key_val(v, c * _L + lane, descending=True)
            gv0, gv1, gi0, gi1 = _merge_topk(gv0, gv1, gi0, gi1, sv, si)
        gi_ref[pl.ds(0, _L)] = gi0
        gi_ref[pl.ds(_L, _L)] = gi1
        g_scal = [gi0[i] for i in range(_L)] + [gi1[i] for i in range(_L)]
        t0g = gv1[_L - 1]

    with jax.named_scope("sc_group_gather"):
        cp_grp = pltpu.make_async_copy(lat3_hbm.at[wid].at[gi_ref], grp_rows, sem2)
        cp_grp.start()
        def zero1_body(i, c):
            for u in range(4):
                zbuf[pl.ds(i * (4 * _L) + u * _L, _L)] = zeros16
            return c

        lax.fori_loop(0, _Z1 // 4, zero1_body, 0)
        cp_grp.wait()

    with jax.named_scope("sc_phase_b"):
        cnt = jnp.int32(0)
        for k in range(_K):
            g = g_scal[k]

            def chunk_body(c, cnt, k=k, g=g):
                v = grp_rows[k, pl.ds(c * _L, _L)]
                m = v >= t0g
                iv = g * _GSZ + c * _L + lane
                plsc.store_compressed(cand_v.at[pl.ds(cnt, _L)], v, mask=m)
                plsc.store_compressed(cand_i.at[pl.ds(cnt, _L)], iv, mask=m)
                return cnt + plsc.all_reduce_population_count(m)[0]

            cnt = lax.fori_loop(0, _GSZ // _L, chunk_body, cnt)

        def p2_body(i, carry):
            kv0, kv1, ki0, ki1 = carry
            v = cand_v[pl.ds(i * _L, _L)]
            iv = cand_i[pl.ds(i * _L, _L)]
            valid = (i * _L + lane) < cnt
            v = jnp.where(valid, v, neg_inf16)
            sv, si = plsc.sort_key_val(v, iv, descending=True)
            return _merge_topk(kv0, kv1, ki0, ki1, sv, si)

        nv = (cnt + _L - 1) // _L
        kv0, kv1, ki0, ki1 = lax.fori_loop(
            0, nv, p2_body, (neg_inf16, neg_inf16, izeros16, izeros16))

    with jax.named_scope("sc_scatter"):
        ki_ref[pl.ds(0, _L)] = ki0
        ki_ref[pl.ds(_L, _L)] = ki1

        cp_w = pltpu.make_async_copy(w_hbm.at[ki_ref], rows_v, sem3)
        cp_w.start()

        def zero2_body(i, c):
            base = _Z1 * _L + i * (4 * _L)
            for u in range(4):
                zbuf[pl.ds(base + u * _L, _L)] = zeros16
            return c

        lax.fori_loop(0, (_LATENT_DIM // _L - _Z1) // 4, zero2_body, 0)
        plsc.store_scatter(zbuf, [ki0], kv0)
        plsc.store_scatter(zbuf, [ki1], kv1)

    with jax.named_scope("sc_gather_wait"):
        cp_pb.wait()
        cp_w.wait()
        cp_sparse = pltpu.make_async_copy(zbuf, sparse_hbm.at[wid], sem0)
        cp_sparse.start()

    with jax.named_scope("sc_decode"):
        vals = [kv0[i] for i in range(_L)] + [kv1[i] for i in range(_L)]

        def col_body(jj, c):
            col = jj * (2 * _L)
            for u in range(2):
                cu = col + u * _L
                p = [None] * 4
                for k in range(_K):
                    term = vals[k] * rows_v[k, pl.ds(cu, _L)]
                    q = k & 3
                    p[q] = term if p[q] is None else p[q] + term
                outv[pl.ds(cu, _L)] = (p[0] + p[1]) + (p[2] + p[3]) + pbv[pl.ds(cu, _L)]
            return c

        lax.fori_loop(0, _INPUT_DIM // (2 * _L), col_body, 0)

    with jax.named_scope("sc_dma_out"):
        pltpu.sync_copy(outv, out_hbm.at[wid])
        cp_sparse.wait()


def _sc_topk_decode(lat3, gmax, w, pre_bias):
    mesh = plsc.VectorSubcoreMesh(
        core_axis_name="c", subcore_axis_name="s",
        num_cores=_NC, num_subcores=_NS,
    )
    f = pl.kernel(
        _sc_body,
        out_type=[
            jax.ShapeDtypeStruct((_BATCH, _INPUT_DIM), jnp.float32),
            jax.ShapeDtypeStruct((_BATCH, _LATENT_DIM), jnp.float32),
        ],
        mesh=mesh,
        compiler_params=pltpu.CompilerParams(needs_layout_passes=False),
        scratch_types=[
            pltpu.VMEM((_LATENT_DIM,), jnp.float32),
            pltpu.VMEM((_K, _GSZ), jnp.float32),
            pltpu.VMEM((_K, _INPUT_DIM), jnp.float32),
            pltpu.VMEM((_INPUT_DIM,), jnp.float32),
            pltpu.VMEM((_INPUT_DIM,), jnp.float32),
            pltpu.VMEM((_NBLK, 1, _GPB), jnp.float32),
            pltpu.VMEM((_K * _GSZ + _L,), jnp.float32),
            pltpu.VMEM((_K * _GSZ + _L,), jnp.int32),
            pltpu.VMEM((2 * _L,), jnp.int32),
            pltpu.VMEM((2 * _L,), jnp.int32),
            pltpu.SemaphoreType.DMA,
            pltpu.SemaphoreType.DMA,
            pltpu.SemaphoreType.DMA,
            pltpu.SemaphoreType.DMA,
        ],
    )
    return f(lat3, gmax, w, pre_bias)


@jax.jit
def kernel(x, W_enc, b_enc, pre_bias):
    b2 = b_enc.reshape(_NBLK, 1, _LBLK)
    lat3, gmax4 = _encode(x, W_enc, b2)
    output, sparse_latents = _sc_topk_decode(lat3, gmax4, W_enc, pre_bias)
    return (output, sparse_latents)
